# baseline (speedup 1.0000x reference)
import functools

import jax
import jax.numpy as jnp
from jax import lax
from jax.experimental import pallas as pl
from jax.experimental.pallas import tpu as pltpu
from jax.experimental.pallas import tpu_sc as plsc

_NC = 2
_NS = 16
_NW = _NC * _NS
_TILE = 1024


def _router_body(x_ref, wr_ref, br_ref, e0_ref, e1_ref, w0_ref, w1_ref):
    x = x_ref[...]
    logits = jnp.dot(x, wr_ref[...], preferred_element_type=jnp.float32) + br_ref[...]
    m1 = jnp.max(logits, axis=-1)
    a1 = jnp.argmax(logits, axis=-1)
    lane = lax.broadcasted_iota(jnp.int32, logits.shape, 1)
    masked = jnp.where(lane == a1[:, None], -jnp.inf, logits)
    m2 = jnp.max(masked, axis=-1)
    a2 = jnp.argmax(masked, axis=-1)
    w0 = 1.0 / (1.0 + jnp.exp(m2 - m1))
    w1 = 1.0 - w0
    shp = logits.shape
    e0_ref[...] = jnp.broadcast_to(a1[:, None], shp)
    e1_ref[...] = jnp.broadcast_to(a2[:, None], shp)
    w0_ref[...] = jnp.broadcast_to(w0[:, None], shp)
    w1_ref[...] = jnp.broadcast_to(w1[:, None], shp)


def _meta_body(E, NTP, e0_ref, e1_ref, pos0_ref, pos1_ref, texp_ref):
    f32 = jnp.float32
    R, C = e0_ref.shape
    e0 = e0_ref[...]
    e1 = e1_ref[...]
    i32 = jnp.int32
    ir = lax.broadcasted_iota(i32, (C, C), 0)
    ic = lax.broadcasted_iota(i32, (C, C), 1)
    U = (ir <= ic).astype(f32)
    rr = lax.broadcasted_iota(i32, (R, R), 0)
    rc = lax.broadcasted_iota(i32, (R, R), 1)
    Lst = (rc < rr).astype(f32)
    pos0 = jnp.zeros((R, C), f32)
    pos1 = jnp.zeros((R, C), f32)
    acc = jnp.zeros((8, NTP), f32)
    lane = lax.broadcasted_iota(i32, (8, NTP), 1).astype(f32)
    start = 0.0
    for e in range(E):
        M0 = (e0 == e).astype(f32)
        M1 = (e1 == e).astype(f32)
        inc0 = jnp.dot(M0, U, preferred_element_type=f32)
        inc1 = jnp.dot(M1, U, preferred_element_type=f32)
        off0 = jnp.dot(Lst, jnp.sum(M0, axis=1, keepdims=True), preferred_element_type=f32)
        off1 = jnp.dot(Lst, jnp.sum(M1, axis=1, keepdims=True), preferred_element_type=f32)
        c0 = jnp.sum(M0)
        rank0 = inc0 - M0 + off0
        rank1 = inc1 - M1 + off1 + c0
        pos0 = pos0 + M0 * (start + rank0)
        pos1 = pos1 + M1 * (start + rank1)
        acc = acc + (lane >= start / float(_TILE)).astype(f32)
        ce = c0 + jnp.sum(M1)
        start = start + jnp.floor((ce + float(_TILE - 1)) / float(_TILE)) * float(_TILE)
    pos0_ref[...] = pos0.astype(jnp.int32)
    pos1_ref[...] = pos1.astype(jnp.int32)
    texp_ref[...] = (acc - 1.0).astype(jnp.int32)


def _ffn_body(texp_ref, xs_ref, w1_ref, b1_ref, w2_ref, b2_ref, ys_ref):
    xb = xs_ref[...]
    h = jnp.maximum(jnp.dot(xb, w1_ref[0], preferred_element_type=jnp.float32) + b1_ref[0], 0.0)
    ys_ref[...] = jnp.dot(h, w2_ref[0], preferred_element_type=jnp.float32) + b2_ref[0]


def _combine_body(ya_ref, w0_ref, w1_ref, out_ref):
    D = out_ref.shape[1]
    ya = ya_ref[...]
    out_ref[...] = ya[:, :D] * w0_ref[:, 0:1] + ya[:, D:] * w1_ref[:, 0:1]


def _sc_dispatch_call(xf, p0, p1, nt):
    T, D = xf.shape
    dt = xf.dtype
    per = T // _NW
    sub = min(64, per)
    nit = per // sub
    mesh = plsc.VectorSubcoreMesh(core_axis_name="c", subcore_axis_name="s", num_cores=_NC)

    def body(x_hbm, p0_hbm, p1_hbm, xs_hbm, idx0_v, idx1_v, xbuf, sem):
        wid = lax.axis_index("s") * _NC + lax.axis_index("c")
        base = wid * per
        for it in range(nit):
            off = base + it * sub
            pltpu.sync_copy(p0_hbm.at[pl.ds(off, sub)], idx0_v)
            pltpu.sync_copy(p1_hbm.at[pl.ds(off, sub)], idx1_v)
            pltpu.sync_copy(x_hbm.at[pl.ds(off, sub)], xbuf)
            c1 = pltpu.async_copy(xbuf, xs_hbm.at[idx0_v], sem)
            c2 = pltpu.async_copy(xbuf, xs_hbm.at[idx1_v], sem)
            c1.wait()
            c2.wait()

    f = pl.kernel(
        body,
        mesh=mesh,
        out_type=jax.ShapeDtypeStruct((nt * _TILE, D), dt),
        scratch_types=[
            pltpu.VMEM((sub,), jnp.int32),
            pltpu.VMEM((sub,), jnp.int32),
            pltpu.VMEM((sub, D), dt),
            pltpu.SemaphoreType.DMA,
        ],
    )
    return f(xf, p0, p1)


def _sc_gather_call(ys, pI):
    P, D = ys.shape
    A = pI.shape[0]
    per = A // _NW
    sub = min(64, per)
    nit = per // sub
    mesh = plsc.VectorSubcoreMesh(core_axis_name="c", subcore_axis_name="s", num_cores=_NC)

    def body(ys_hbm, pI_hbm, ya_hbm, idx_v, rows_v, sem):
        wid = lax.axis_index("s") * _NC + lax.axis_index("c")
        base = wid * per
        for it in range(nit):
            off = base + it * sub
            pltpu.sync_copy(pI_hbm.at[pl.ds(off, sub)], idx_v)
            pltpu.async_copy(ys_hbm.at[idx_v], rows_v, sem).wait()
            pltpu.sync_copy(rows_v, ya_hbm.at[pl.ds(off, sub)])

    f = pl.kernel(
        body,
        mesh=mesh,
        out_type=jax.ShapeDtypeStruct((A, D), jnp.float32),
        scratch_types=[
            pltpu.VMEM((sub,), jnp.int32),
            pltpu.VMEM((sub, D), jnp.float32),
            pltpu.SemaphoreType.DMA,
        ],
    )
    return f(ys, pI)


def kernel(x, Wr, br, W1, b1, W2, b2):
    B, S, D = x.shape
    E = Wr.shape[1]
    FF = W1.shape[2]
    T = B * S
    A = 2 * T
    NT = (A + (E - 1) * (_TILE - 1)) // _TILE
    NTP = ((NT + 127) // 128) * 128
    EP = 128

    xf = x.reshape(T, D)
    Wr_p = jnp.pad(Wr, ((0, 0), (0, EP - E)))
    br_p = jnp.concatenate([br, jnp.full((EP - E,), -1e30, br.dtype)]).reshape(1, EP)
    b1r = b1.reshape(E, 1, FF)
    b2r = b2.reshape(E, 1, D)

    TMr = min(1024, T)
    e0b, e1b, w0b, w1b = pl.pallas_call(
        _router_body,
        grid=(T // TMr,),
        in_specs=[
            pl.BlockSpec((TMr, D), lambda i: (i, 0)),
            pl.BlockSpec((D, EP), lambda i: (0, 0)),
            pl.BlockSpec((1, EP), lambda i: (0, 0)),
        ],
        out_specs=[pl.BlockSpec((TMr, EP), lambda i: (i, 0))] * 4,
        out_shape=[
            jax.ShapeDtypeStruct((T, EP), jnp.int32),
            jax.ShapeDtypeStruct((T, EP), jnp.int32),
            jax.ShapeDtypeStruct((T, EP), jnp.float32),
            jax.ShapeDtypeStruct((T, EP), jnp.float32),
        ],
    )(xf, Wr_p, br_p)

    e0 = e0b[:, 0].reshape(T // 128, 128)
    e1 = e1b[:, 0].reshape(T // 128, 128)
    pos0m, pos1m, texpm = pl.pallas_call(
        functools.partial(_meta_body, E, NTP),
        out_shape=[
            jax.ShapeDtypeStruct((T // 128, 128), jnp.int32),
            jax.ShapeDtypeStruct((T // 128, 128), jnp.int32),
            jax.ShapeDtypeStruct((8, NTP), jnp.int32),
        ],
    )(e0, e1)
    pos0 = pos0m.reshape(T)
    pos1 = pos1m.reshape(T)
    texp = texpm[0, :NT]
    posI = jnp.stack([pos0, pos1], axis=1).reshape(A)

    xs = _sc_dispatch_call(xf, pos0, pos1, NT)

    grid_spec = pltpu.PrefetchScalarGridSpec(
        num_scalar_prefetch=1,
        grid=(NT,),
        in_specs=[
            pl.BlockSpec((_TILE, D), lambda i, t: (i, 0)),
            pl.BlockSpec((1, D, FF), lambda i, t: (t[i], 0, 0)),
            pl.BlockSpec((1, 1, FF), lambda i, t: (t[i], 0, 0)),
            pl.BlockSpec((1, FF, D), lambda i, t: (t[i], 0, 0)),
            pl.BlockSpec((1, 1, D), lambda i, t: (t[i], 0, 0)),
        ],
        out_specs=pl.BlockSpec((_TILE, D), lambda i, t: (i, 0)),
    )
    ys = pl.pallas_call(
        _ffn_body,
        grid_spec=grid_spec,
        out_shape=jax.ShapeDtypeStruct((NT * _TILE, D), jnp.float32),
    )(texp, xs, W1, b1r, W2, b2r)

    ya = _sc_gather_call(ys, posI)

    TMc = min(1024, T)
    out = pl.pallas_call(
        _combine_body,
        grid=(T // TMc,),
        in_specs=[
            pl.BlockSpec((TMc, 2 * D), lambda i: (i, 0)),
            pl.BlockSpec((TMc, EP), lambda i: (i, 0)),
            pl.BlockSpec((TMc, EP), lambda i: (i, 0)),
        ],
        out_specs=pl.BlockSpec((TMc, D), lambda i: (i, 0)),
        out_shape=jax.ShapeDtypeStruct((T, D), jnp.float32),
    )(ya.reshape(T, 2 * D), w0b, w1b)

    return out.reshape(B, S, D)


# speedup vs baseline: 1.0118x; 1.0118x over previous
import functools

import jax
import jax.numpy as jnp
from jax import lax
from jax.experimental import pallas as pl
from jax.experimental.pallas import tpu as pltpu
from jax.experimental.pallas import tpu_sc as plsc

_NC = 2
_NS = 16
_NW = _NC * _NS
_TILE = 512


def _router_body(x_ref, wr_ref, br_ref, e0_ref, e1_ref, w0_ref, w1_ref):
    x = x_ref[...]
    logits = jnp.dot(x, wr_ref[...], preferred_element_type=jnp.float32) + br_ref[...]
    m1 = jnp.max(logits, axis=-1)
    a1 = jnp.argmax(logits, axis=-1)
    lane = lax.broadcasted_iota(jnp.int32, logits.shape, 1)
    masked = jnp.where(lane == a1[:, None], -jnp.inf, logits)
    m2 = jnp.max(masked, axis=-1)
    a2 = jnp.argmax(masked, axis=-1)
    w0 = 1.0 / (1.0 + jnp.exp(m2 - m1))
    w1 = 1.0 - w0
    shp = logits.shape
    e0_ref[...] = jnp.broadcast_to(a1[:, None], shp)
    e1_ref[...] = jnp.broadcast_to(a2[:, None], shp)
    w0_ref[...] = jnp.broadcast_to(w0[:, None], shp)
    w1_ref[...] = jnp.broadcast_to(w1[:, None], shp)


def _meta_body(E, NTP, e0_ref, e1_ref, pos0_ref, pos1_ref, texp_ref):
    f32 = jnp.float32
    R, C = e0_ref.shape
    e0 = e0_ref[...]
    e1 = e1_ref[...]
    i32 = jnp.int32
    ir = lax.broadcasted_iota(i32, (C, C), 0)
    ic = lax.broadcasted_iota(i32, (C, C), 1)
    U = (ir <= ic).astype(f32)
    rr = lax.broadcasted_iota(i32, (R, R), 0)
    rc = lax.broadcasted_iota(i32, (R, R), 1)
    Lst = (rc < rr).astype(f32)
    pos0 = jnp.zeros((R, C), f32)
    pos1 = jnp.zeros((R, C), f32)
    acc = jnp.zeros((8, NTP), f32)
    lane = lax.broadcasted_iota(i32, (8, NTP), 1).astype(f32)
    start = 0.0
    for e in range(E):
        M0 = (e0 == e).astype(f32)
        M1 = (e1 == e).astype(f32)
        inc0 = jnp.dot(M0, U, preferred_element_type=f32)
        inc1 = jnp.dot(M1, U, preferred_element_type=f32)
        off0 = jnp.dot(Lst, jnp.sum(M0, axis=1, keepdims=True), preferred_element_type=f32)
        off1 = jnp.dot(Lst, jnp.sum(M1, axis=1, keepdims=True), preferred_element_type=f32)
        c0 = jnp.sum(M0)
        rank0 = inc0 - M0 + off0
        rank1 = inc1 - M1 + off1 + c0
        pos0 = pos0 + M0 * (start + rank0)
        pos1 = pos1 + M1 * (start + rank1)
        acc = acc + (lane >= start / float(_TILE)).astype(f32)
        ce = c0 + jnp.sum(M1)
        start = start + jnp.floor((ce + float(_TILE - 1)) / float(_TILE)) * float(_TILE)
    pos0_ref[...] = pos0.astype(jnp.int32)
    pos1_ref[...] = pos1.astype(jnp.int32)
    texp_ref[...] = (acc - 1.0).astype(jnp.int32)


def _ffn_body(texp_ref, xs_ref, w1_ref, b1_ref, w2_ref, b2_ref, ys_ref):
    xb = xs_ref[...]
    h = jnp.maximum(jnp.dot(xb, w1_ref[0], preferred_element_type=jnp.float32) + b1_ref[0], 0.0)
    ys_ref[...] = jnp.dot(h, w2_ref[0], preferred_element_type=jnp.float32) + b2_ref[0]


def _combine_body(ya_ref, w0_ref, w1_ref, out_ref):
    D = out_ref.shape[1]
    ya = ya_ref[...]
    out_ref[...] = ya[:, :D] * w0_ref[:, 0:1] + ya[:, D:] * w1_ref[:, 0:1]


def _sc_dispatch_call(xf, p0, p1, nt):
    T, D = xf.shape
    dt = xf.dtype
    per = T // _NW
    if per % 96 == 64:
        chunks = [96] * (per // 96) + [64]
    elif per % 64 == 0:
        chunks = [64] * (per // 64) if per >= 64 else [per]
    else:
        chunks = [per]
    big = max(chunks)
    mesh = plsc.VectorSubcoreMesh(core_axis_name="c", subcore_axis_name="s", num_cores=_NC)

    def body(x_hbm, p0_hbm, p1_hbm, xs_hbm, idx0_b, idx1_b, idx0_s, idx1_s, xbuf, sem):
        wid = lax.axis_index("s") * _NC + lax.axis_index("c")
        base = wid * per
        off = base
        for sub in chunks:
            idx0_v = idx0_b if sub == big else idx0_s
            idx1_v = idx1_b if sub == big else idx1_s
            xsrc = xbuf if sub == big else xbuf.at[pl.ds(0, sub)]
            pltpu.sync_copy(p0_hbm.at[pl.ds(off, sub)], idx0_v)
            pltpu.sync_copy(p1_hbm.at[pl.ds(off, sub)], idx1_v)
            pltpu.sync_copy(x_hbm.at[pl.ds(off, sub)], xsrc)
            c1 = pltpu.async_copy(xsrc, xs_hbm.at[idx0_v], sem)
            c2 = pltpu.async_copy(xsrc, xs_hbm.at[idx1_v], sem)
            c1.wait()
            c2.wait()
            off = off + sub

    small = min(chunks)
    f = pl.kernel(
        body,
        mesh=mesh,
        out_type=jax.ShapeDtypeStruct((nt * _TILE, D), dt),
        scratch_types=[
            pltpu.VMEM((big,), jnp.int32),
            pltpu.VMEM((big,), jnp.int32),
            pltpu.VMEM((small,), jnp.int32),
            pltpu.VMEM((small,), jnp.int32),
            pltpu.VMEM((big, D), dt),
            pltpu.SemaphoreType.DMA,
        ],
    )
    return f(xf, p0, p1)


def _sc_gather_call(ys, pI):
    P, D = ys.shape
    A = pI.shape[0]
    per = A // _NW
    if per % 96 == 32:
        chunks = [96] * (per // 96) + [32]
    elif per % 64 == 0:
        chunks = [64] * (per // 64) if per >= 64 else [per]
    else:
        chunks = [per]
    big = max(chunks)
    small = min(chunks)
    mesh = plsc.VectorSubcoreMesh(core_axis_name="c", subcore_axis_name="s", num_cores=_NC)

    def body(ys_hbm, pI_hbm, ya_hbm, idx_b, idx_s, rows_v, sem):
        wid = lax.axis_index("s") * _NC + lax.axis_index("c")
        base = wid * per
        off = base
        for sub in chunks:
            idx_v = idx_b if sub == big else idx_s
            rdst = rows_v if sub == big else rows_v.at[pl.ds(0, sub)]
            pltpu.sync_copy(pI_hbm.at[pl.ds(off, sub)], idx_v)
            pltpu.async_copy(ys_hbm.at[idx_v], rdst, sem).wait()
            pltpu.sync_copy(rdst, ya_hbm.at[pl.ds(off, sub)])
            off = off + sub

    f = pl.kernel(
        body,
        mesh=mesh,
        out_type=jax.ShapeDtypeStruct((A, D), jnp.float32),
        scratch_types=[
            pltpu.VMEM((big,), jnp.int32),
            pltpu.VMEM((small,), jnp.int32),
            pltpu.VMEM((big, D), jnp.float32),
            pltpu.SemaphoreType.DMA,
        ],
    )
    return f(ys, pI)


def kernel(x, Wr, br, W1, b1, W2, b2):
    B, S, D = x.shape
    E = Wr.shape[1]
    FF = W1.shape[2]
    T = B * S
    A = 2 * T
    NT = (A + (E - 1) * (_TILE - 1)) // _TILE
    NTP = ((NT + 127) // 128) * 128
    EP = 128

    xf = x.reshape(T, D)
    Wr_p = jnp.pad(Wr, ((0, 0), (0, EP - E)))
    br_p = jnp.concatenate([br, jnp.full((EP - E,), -1e30, br.dtype)]).reshape(1, EP)
    b1r = b1.reshape(E, 1, FF)
    b2r = b2.reshape(E, 1, D)

    TMr = min(1024, T)
    e0b, e1b, w0b, w1b = pl.pallas_call(
        _router_body,
        grid=(T // TMr,),
        in_specs=[
            pl.BlockSpec((TMr, D), lambda i: (i, 0)),
            pl.BlockSpec((D, EP), lambda i: (0, 0)),
            pl.BlockSpec((1, EP), lambda i: (0, 0)),
        ],
        out_specs=[pl.BlockSpec((TMr, EP), lambda i: (i, 0))] * 4,
        out_shape=[
            jax.ShapeDtypeStruct((T, EP), jnp.int32),
            jax.ShapeDtypeStruct((T, EP), jnp.int32),
            jax.ShapeDtypeStruct((T, EP), jnp.float32),
            jax.ShapeDtypeStruct((T, EP), jnp.float32),
        ],
    )(xf, Wr_p, br_p)

    e0 = e0b[:, 0].reshape(T // 128, 128)
    e1 = e1b[:, 0].reshape(T // 128, 128)
    pos0m, pos1m, texpm = pl.pallas_call(
        functools.partial(_meta_body, E, NTP),
        out_shape=[
            jax.ShapeDtypeStruct((T // 128, 128), jnp.int32),
            jax.ShapeDtypeStruct((T // 128, 128), jnp.int32),
            jax.ShapeDtypeStruct((8, NTP), jnp.int32),
        ],
    )(e0, e1)
    pos0 = pos0m.reshape(T)
    pos1 = pos1m.reshape(T)
    texp = texpm[0, :NT]
    posI = jnp.stack([pos0, pos1], axis=1).reshape(A)

    xs = _sc_dispatch_call(xf, pos0, pos1, NT)

    grid_spec = pltpu.PrefetchScalarGridSpec(
        num_scalar_prefetch=1,
        grid=(NT,),
        in_specs=[
            pl.BlockSpec((_TILE, D), lambda i, t: (i, 0)),
            pl.BlockSpec((1, D, FF), lambda i, t: (t[i], 0, 0)),
            pl.BlockSpec((1, 1, FF), lambda i, t: (t[i], 0, 0)),
            pl.BlockSpec((1, FF, D), lambda i, t: (t[i], 0, 0)),
            pl.BlockSpec((1, 1, D), lambda i, t: (t[i], 0, 0)),
        ],
        out_specs=pl.BlockSpec((_TILE, D), lambda i, t: (i, 0)),
    )
    ys = pl.pallas_call(
        _ffn_body,
        grid_spec=grid_spec,
        out_shape=jax.ShapeDtypeStruct((NT * _TILE, D), jnp.float32),
    )(texp, xs, W1, b1r, W2, b2r)

    ya = _sc_gather_call(ys, posI)

    TMc = min(1024, T)
    out = pl.pallas_call(
        _combine_body,
        grid=(T // TMc,),
        in_specs=[
            pl.BlockSpec((TMc, 2 * D), lambda i: (i, 0)),
            pl.BlockSpec((TMc, EP), lambda i: (i, 0)),
            pl.BlockSpec((TMc, EP), lambda i: (i, 0)),
        ],
        out_specs=pl.BlockSpec((TMc, D), lambda i: (i, 0)),
        out_shape=jax.ShapeDtypeStruct((T, D), jnp.float32),
    )(ya.reshape(T, 2 * D), w0b, w1b)

    return out.reshape(B, S, D)
